# baseline (speedup 1.0000x reference)
import functools

import jax
import jax.numpy as jnp
from jax import lax
from jax.experimental import pallas as pl
from jax.experimental.pallas import tpu as pltpu
from jax.experimental.pallas import tpu_sc as plsc

N = 10000
NPAD = 10240
E_RAW = 640000
E = E_RAW + N
NW = 32
K = 112
EPT = 20384
NCH = EPT // K
NCHP = NCH + 1
EPAD = NW * EPT
WR = 144
WN = 16
RPT = NPAD // 16
ZR = 16
BLK = 512

_f32 = jnp.float32
_i32 = jnp.int32


def _sc_body(width, T_hbm, D_hbm, e_hbm, out_hbm,
             acc, idx_a, idx_b, rows_a, rows_b, adv_a, adv_b,
             zbuf, ga, gb, da, db, sa, sb):
    c = lax.axis_index("c")
    s = lax.axis_index("s")
    wid = s * 2 + c
    iota16 = lax.iota(_i32, 16)
    z16 = jnp.zeros((16,), _f32)
    zi16 = jnp.zeros((16,), _i32)
    ng = width // 16
    idxs = [idx_a, idx_b]
    rows = [rows_a, rows_b]
    advs = [adv_a, adv_b]
    gsem = [ga, gb]
    dsem = [da, db]
    ssem = [sa, sb]

    for r in range(ZR):
        for q in range(ng):
            zbuf[r, pl.ds(q * 16, 16)] = z16
    for bb in range(RPT // ZR):
        pltpu.sync_copy(zbuf, acc.at[pl.ds(s * RPT + bb * ZR, ZR)])
    plsc.subcore_barrier()

    c128 = jnp.full((16,), 128, _i32)
    c129 = jnp.full((16,), 129, _i32)
    c1 = jnp.full((16,), 1, _i32)

    def compute(b):
        rows_v = rows[b]
        adv = advs[b]
        if width == WR:
            for g in range(K // 16):
                ids = iota16 + g * 16
                as0 = plsc.load_gather(rows_v, [ids, c128])
                as1 = plsc.load_gather(rows_v, [ids, c129])
                ad0 = plsc.load_gather(adv, [ids, zi16])
                ad1 = plsc.load_gather(adv, [ids, c1])
                s0 = as0 + ad0
                s1 = as1 + ad1
                w0 = jnp.exp(jnp.where(s0 > 0, s0, 0.2 * s0))
                w1 = jnp.exp(jnp.where(s1 > 0, s1, 0.2 * s1))
                for j in range(16):
                    e = g * 16 + j
                    w0b = jnp.full((16,), w0[j], _f32)
                    w1b = jnp.full((16,), w1[j], _f32)
                    for q in range(4):
                        rows_v[e, pl.ds(q * 16, 16)] = (
                            rows_v[e, pl.ds(q * 16, 16)] * w0b)
                    for q in range(4, 8):
                        rows_v[e, pl.ds(q * 16, 16)] = (
                            rows_v[e, pl.ds(q * 16, 16)] * w1b)
                    rows_v[e, pl.ds(128, 16)] = jnp.where(
                        iota16 == 0, w0b, jnp.where(iota16 == 1, w1b, z16))
        else:
            for g in range(K // 16):
                ids = iota16 + g * 16
                hv = plsc.load_gather(rows_v, [ids, zi16])
                asv = plsc.load_gather(rows_v, [ids, c1])
                adv0 = plsc.load_gather(adv, [ids, zi16])
                s0 = asv + adv0
                w = jnp.exp(jnp.where(s0 > 0, s0, 0.2 * s0))
                nm = w * hv
                for j in range(16):
                    e = g * 16 + j
                    nb = jnp.full((16,), nm[j], _f32)
                    wb = jnp.full((16,), w[j], _f32)
                    rows_v[e, pl.ds(0, 16)] = jnp.where(
                        iota16 == 0, nb, jnp.where(iota16 == 1, wb, z16))

    for q in range(K // 16):
        idx_b[1, pl.ds(q * 16, 16)] = zi16
    for r in range(K):
        for q in range(ng):
            rows_b[r, pl.ds(q * 16, 16)] = z16
    pltpu.async_copy(rows_b, acc.at[idx_b.at[1]], sb, add=True)
    pltpu.sync_copy(e_hbm.at[wid, 0], idx_a)
    pltpu.async_copy(T_hbm.at[idx_a.at[0]], rows_a, ga)
    pltpu.async_copy(D_hbm.at[idx_a.at[1]], adv_a, da)

    def pair(p, carry):
        for b in (0, 1):
            ci = 2 * p + b
            nb = 1 - b
            pltpu.make_async_copy(rows[nb], acc.at[idxs[nb].at[1]],
                                  ssem[nb]).wait()
            pltpu.sync_copy(e_hbm.at[wid, ci + 1], idxs[nb])
            pltpu.async_copy(T_hbm.at[idxs[nb].at[0]], rows[nb], gsem[nb])
            pltpu.async_copy(D_hbm.at[idxs[nb].at[1]], advs[nb], dsem[nb])
            pltpu.make_async_copy(T_hbm.at[idxs[b].at[0]], rows[b],
                                  gsem[b]).wait()
            pltpu.make_async_copy(D_hbm.at[idxs[b].at[1]], advs[b],
                                  dsem[b]).wait()
            compute(b)
            pltpu.async_copy(rows[b], acc.at[idxs[b].at[1]], ssem[b], add=True)
        return carry

    lax.fori_loop(0, NCH // 2, pair, 0)
    pltpu.make_async_copy(T_hbm.at[idx_a.at[0]], rows_a, ga).wait()
    pltpu.make_async_copy(D_hbm.at[idx_a.at[1]], adv_a, da).wait()
    pltpu.make_async_copy(rows_b, acc.at[idx_b.at[1]], sb).wait()
    plsc.subcore_barrier()
    pltpu.sync_copy(acc.at[pl.ds(s * RPT, RPT)],
                    out_hbm.at[c, pl.ds(s * RPT, RPT)])


@functools.lru_cache(maxsize=None)
def _make_sc_pass(width):
    mesh = plsc.VectorSubcoreMesh(core_axis_name="c", subcore_axis_name="s")
    return functools.partial(
        pl.kernel,
        out_type=jax.ShapeDtypeStruct((2, NPAD, width), _f32),
        mesh=mesh,
        scratch_types=[
            pltpu.VMEM_SHARED((NPAD, width), _f32),
            pltpu.VMEM((2, K), _i32),
            pltpu.VMEM((2, K), _i32),
            pltpu.VMEM((K, width), _f32),
            pltpu.VMEM((K, width), _f32),
            pltpu.VMEM((K, 16), _f32),
            pltpu.VMEM((K, 16), _f32),
            pltpu.VMEM((ZR, width), _f32),
            pltpu.SemaphoreType.DMA,
            pltpu.SemaphoreType.DMA,
            pltpu.SemaphoreType.DMA,
            pltpu.SemaphoreType.DMA,
            pltpu.SemaphoreType.DMA,
            pltpu.SemaphoreType.DMA,
        ],
        compiler_params=pltpu.CompilerParams(use_tc_tiling_on_sc=False,
                                             needs_layout_passes=False),
    )(functools.partial(_sc_body, width))


def _sc_wide(t, d, e3):
    return _make_sc_pass(WR)(t, d, e3)


def _sc_narrow(t, d, e3):
    return _make_sc_pass(WN)(t, d, e3)


def _prep1_body(x_ref, w_ref, as_ref, ad_ref, t_ref, d_ref):
    h = x_ref[...] * w_ref[...]
    t = h * as_ref[...]
    s0 = jnp.sum(t[:, :64], axis=1, keepdims=True)
    s1 = jnp.sum(t[:, 64:], axis=1, keepdims=True)
    u = h * ad_ref[...]
    d0 = jnp.sum(u[:, :64], axis=1, keepdims=True)
    d1 = jnp.sum(u[:, 64:], axis=1, keepdims=True)
    zpad = jnp.zeros((h.shape[0], 14), _f32)
    t_ref[...] = jnp.concatenate([h, s0, s1, zpad], axis=1)
    d_ref[...] = jnp.concatenate([d0, d1, zpad], axis=1)


def _tc_prep1(x_pad, w1, asf, adf):
    grid = (NPAD // BLK,)
    return pl.pallas_call(
        _prep1_body,
        grid=grid,
        in_specs=[
            pl.BlockSpec((BLK, 1), lambda i: (i, 0)),
            pl.BlockSpec((1, 128), lambda i: (0, 0)),
            pl.BlockSpec((1, 128), lambda i: (0, 0)),
            pl.BlockSpec((1, 128), lambda i: (0, 0)),
        ],
        out_specs=[
            pl.BlockSpec((BLK, WR), lambda i: (i, 0)),
            pl.BlockSpec((BLK, WN), lambda i: (i, 0)),
        ],
        out_shape=[
            jax.ShapeDtypeStruct((NPAD, WR), _f32),
            jax.ShapeDtypeStruct((NPAD, WN), _f32),
        ],
    )(x_pad, w1, asf, adf)


def _combine_body(p0_ref, p1_ref, b_ref, w_ref, as_ref, ad_ref, t_ref, d_ref):
    p0 = p0_ref[...]
    p1 = p1_ref[...]
    num = p0[:, :128] + p1[:, :128]
    den0 = p0[:, 128:129] + p1[:, 128:129]
    den1 = p0[:, 129:130] + p1[:, 129:130]
    B = num.shape[0]
    den = jnp.concatenate([jnp.broadcast_to(den0, (B, 64)),
                           jnp.broadcast_to(den1, (B, 64))], axis=1)
    y = num / (den + 1e-16) + b_ref[...]
    y = jnp.maximum(y, 0.0)
    h = jnp.dot(y, w_ref[...], preferred_element_type=_f32)
    t = h * as_ref[...]
    s0 = jnp.sum(t[:, :64], axis=1, keepdims=True)
    s1 = jnp.sum(t[:, 64:], axis=1, keepdims=True)
    u = h * ad_ref[...]
    d0 = jnp.sum(u[:, :64], axis=1, keepdims=True)
    d1 = jnp.sum(u[:, 64:], axis=1, keepdims=True)
    zpad = jnp.zeros((B, 14), _f32)
    t_ref[...] = jnp.concatenate([h, s0, s1, zpad], axis=1)
    d_ref[...] = jnp.concatenate([d0, d1, zpad], axis=1)


def _tc_combine2(p0, p1, b1, w2, asf, adf):
    grid = (NPAD // BLK,)
    return pl.pallas_call(
        _combine_body,
        grid=grid,
        in_specs=[
            pl.BlockSpec((BLK, WR), lambda i: (i, 0)),
            pl.BlockSpec((BLK, WR), lambda i: (i, 0)),
            pl.BlockSpec((1, 128), lambda i: (0, 0)),
            pl.BlockSpec((128, 128), lambda i: (0, 0)),
            pl.BlockSpec((1, 128), lambda i: (0, 0)),
            pl.BlockSpec((1, 128), lambda i: (0, 0)),
        ],
        out_specs=[
            pl.BlockSpec((BLK, WR), lambda i: (i, 0)),
            pl.BlockSpec((BLK, WN), lambda i: (i, 0)),
        ],
        out_shape=[
            jax.ShapeDtypeStruct((NPAD, WR), _f32),
            jax.ShapeDtypeStruct((NPAD, WN), _f32),
        ],
    )(p0, p1, b1, w2, asf, adf)


def _combine3_body(p0_ref, p1_ref, b_ref, w_ref, as_ref, ad_ref, t_ref, d_ref):
    p0 = p0_ref[...]
    p1 = p1_ref[...]
    num = p0[:, :128] + p1[:, :128]
    den0 = p0[:, 128:129] + p1[:, 128:129]
    den1 = p0[:, 129:130] + p1[:, 129:130]
    B = num.shape[0]
    den = jnp.concatenate([jnp.broadcast_to(den0, (B, 64)),
                           jnp.broadcast_to(den1, (B, 64))], axis=1)
    y = num / (den + 1e-16) + b_ref[...]
    y = jnp.maximum(y, 0.0)
    h = jnp.dot(y, w_ref[...], preferred_element_type=_f32)
    as_s = as_ref[0, 0]
    ad_s = ad_ref[0, 0]
    t_ref[...] = jnp.concatenate([h, h * as_s, jnp.zeros((B, 14), _f32)],
                                 axis=1)
    d_ref[...] = jnp.concatenate([h * ad_s, jnp.zeros((B, 15), _f32)], axis=1)


def _tc_combine3(p0, p1, b2, w3, as3, ad3):
    grid = (NPAD // BLK,)
    return pl.pallas_call(
        _combine3_body,
        grid=grid,
        in_specs=[
            pl.BlockSpec((BLK, WR), lambda i: (i, 0)),
            pl.BlockSpec((BLK, WR), lambda i: (i, 0)),
            pl.BlockSpec((1, 128), lambda i: (0, 0)),
            pl.BlockSpec((128, 1), lambda i: (0, 0)),
            pl.BlockSpec((1, 1), lambda i: (0, 0)),
            pl.BlockSpec((1, 1), lambda i: (0, 0)),
        ],
        out_specs=[
            pl.BlockSpec((BLK, WN), lambda i: (i, 0)),
            pl.BlockSpec((BLK, WN), lambda i: (i, 0)),
        ],
        out_shape=[
            jax.ShapeDtypeStruct((NPAD, WN), _f32),
            jax.ShapeDtypeStruct((NPAD, WN), _f32),
        ],
    )(p0, p1, b2, w3, as3, ad3)


def _final_body(q0_ref, q1_ref, b_ref, o_ref):
    q0 = q0_ref[...]
    q1 = q1_ref[...]
    num = q0[:, 0:1] + q1[:, 0:1]
    den = q0[:, 1:2] + q1[:, 1:2]
    o_ref[...] = jax.nn.sigmoid(num / (den + 1e-16) + b_ref[...])


def _tc_final(q0, q1, b3):
    grid = (NPAD // BLK,)
    return pl.pallas_call(
        _final_body,
        grid=grid,
        in_specs=[
            pl.BlockSpec((BLK, WN), lambda i: (i, 0)),
            pl.BlockSpec((BLK, WN), lambda i: (i, 0)),
            pl.BlockSpec((1, 1), lambda i: (0, 0)),
        ],
        out_specs=pl.BlockSpec((BLK, 1), lambda i: (i, 0)),
        out_shape=jax.ShapeDtypeStruct((NPAD, 1), _f32),
    )(q0, q1, b3)


def kernel(x, edge_index, batch, W1, as1, ad1, b1, W2, as2, ad2, b2,
           W3, as3, ad3, b3):
    del batch
    loop = jnp.arange(N, dtype=edge_index.dtype)
    pad = jnp.full((EPAD - E,), N, dtype=edge_index.dtype)
    dummy = jnp.full((NW, 1, 2, K), N, dtype=edge_index.dtype)
    src3 = jnp.concatenate([edge_index[0], loop, pad]).reshape(NW, NCH, 1, K)
    dst3 = jnp.concatenate([edge_index[1], loop, pad]).reshape(NW, NCH, 1, K)
    e3 = jnp.concatenate(
        [jnp.concatenate([src3, dst3], axis=2), dummy], axis=1)
    x_pad = jnp.pad(x, ((0, NPAD - N), (0, 0)))

    t1, d1 = _tc_prep1(x_pad, W1.reshape(1, 128), as1.reshape(1, 128),
                       ad1.reshape(1, 128))
    p1 = _sc_wide(t1, d1, e3)
    t2, d2 = _tc_combine2(p1[0], p1[1], b1.reshape(1, 128), W2,
                          as2.reshape(1, 128), ad2.reshape(1, 128))
    p2 = _sc_wide(t2, d2, e3)
    t3, d3 = _tc_combine3(p2[0], p2[1], b2.reshape(1, 128), W3,
                          as3.reshape(1, 1), ad3.reshape(1, 1))
    p3 = _sc_narrow(t3, d3, e3)
    out = _tc_final(p3[0], p3[1], b3.reshape(1, 1))
    return out[:N]


# speedup vs baseline: 1.0427x; 1.0427x over previous
import functools

import jax
import jax.numpy as jnp
from jax import lax
from jax.experimental import pallas as pl
from jax.experimental.pallas import tpu as pltpu
from jax.experimental.pallas import tpu_sc as plsc

N = 10000
NPAD = 10240
E_RAW = 640000
E = E_RAW + N
NW = 32
K = 96
EPT = 20352
NCH = EPT // K
NCHP = NCH + 1
EPAD = NW * EPT
WR = 144
WN = 16
RPT = NPAD // 16
ZR = 40
BLK = 512

_f32 = jnp.float32
_i32 = jnp.int32


def _sc_body(width, T_hbm, D_hbm, e_hbm, out_hbm,
             acc, idx_a, idx_b, rows_a, rows_b, adv_a, adv_b,
             zbuf, ga, gb, da, db, sa, sb, ia, ib):
    c = lax.axis_index("c")
    s = lax.axis_index("s")
    wid = s * 2 + c
    iota16 = lax.iota(_i32, 16)
    z16 = jnp.zeros((16,), _f32)
    zi16 = jnp.zeros((16,), _i32)
    ng = width // 16
    idxs = [idx_a, idx_b]
    rows = [rows_a, rows_b]
    advs = [adv_a, adv_b]
    gsem = [ga, gb]
    dsem = [da, db]
    ssem = [sa, sb]
    isem = [ia, ib]

    for r in range(ZR):
        for q in range(ng):
            zbuf[r, pl.ds(q * 16, 16)] = z16
    for bb in range(RPT // ZR):
        pltpu.sync_copy(zbuf, acc.at[pl.ds(s * RPT + bb * ZR, ZR)])
    plsc.subcore_barrier()

    c128 = jnp.full((16,), 128, _i32)
    c129 = jnp.full((16,), 129, _i32)
    c1 = jnp.full((16,), 1, _i32)

    def compute(b):
        rows_v = rows[b]
        adv = advs[b]
        if width == WR:
            for g in range(K // 16):
                ids = iota16 + g * 16
                as0 = plsc.load_gather(rows_v, [ids, c128])
                as1 = plsc.load_gather(rows_v, [ids, c129])
                ad0 = plsc.load_gather(adv, [ids, zi16])
                ad1 = plsc.load_gather(adv, [ids, c1])
                s0 = as0 + ad0
                s1 = as1 + ad1
                w0 = jnp.exp(jnp.where(s0 > 0, s0, 0.2 * s0))
                w1 = jnp.exp(jnp.where(s1 > 0, s1, 0.2 * s1))
                for j in range(16):
                    e = g * 16 + j
                    w0b = jnp.full((16,), w0[j], _f32)
                    w1b = jnp.full((16,), w1[j], _f32)
                    for q in range(4):
                        rows_v[e, pl.ds(q * 16, 16)] = (
                            rows_v[e, pl.ds(q * 16, 16)] * w0b)
                    for q in range(4, 8):
                        rows_v[e, pl.ds(q * 16, 16)] = (
                            rows_v[e, pl.ds(q * 16, 16)] * w1b)
                    rows_v[e, pl.ds(128, 16)] = jnp.where(
                        iota16 == 0, w0b, jnp.where(iota16 == 1, w1b, z16))
        else:
            for g in range(K // 16):
                ids = iota16 + g * 16
                hv = plsc.load_gather(rows_v, [ids, zi16])
                asv = plsc.load_gather(rows_v, [ids, c1])
                adv0 = plsc.load_gather(adv, [ids, zi16])
                s0 = asv + adv0
                w = jnp.exp(jnp.where(s0 > 0, s0, 0.2 * s0))
                nm = w * hv
                for j in range(16):
                    e = g * 16 + j
                    nb = jnp.full((16,), nm[j], _f32)
                    wb = jnp.full((16,), w[j], _f32)
                    rows_v[e, pl.ds(0, 16)] = jnp.where(
                        iota16 == 0, nb, jnp.where(iota16 == 1, wb, z16))

    for q in range(K // 16):
        idx_b[1, pl.ds(q * 16, 16)] = zi16
    for r in range(K):
        for q in range(ng):
            rows_b[r, pl.ds(q * 16, 16)] = z16
    pltpu.async_copy(rows_b, acc.at[idx_b.at[1]], sb, add=True)
    pltpu.sync_copy(e_hbm.at[wid, 0], idx_a)
    pltpu.async_copy(T_hbm.at[idx_a.at[0]], rows_a, ga)
    pltpu.async_copy(D_hbm.at[idx_a.at[1]], adv_a, da)

    def pair(p, carry):
        for b in (0, 1):
            ci = 2 * p + b
            nb = 1 - b
            pltpu.make_async_copy(rows[nb], acc.at[idxs[nb].at[1]],
                                  ssem[nb]).wait()
            pltpu.async_copy(e_hbm.at[wid, ci + 1], idxs[nb], isem[nb])
            pltpu.make_async_copy(T_hbm.at[idxs[b].at[0]], rows[b],
                                  gsem[b]).wait()
            pltpu.make_async_copy(D_hbm.at[idxs[b].at[1]], advs[b],
                                  dsem[b]).wait()
            pltpu.make_async_copy(e_hbm.at[wid, ci + 1], idxs[nb],
                                  isem[nb]).wait()
            pltpu.async_copy(T_hbm.at[idxs[nb].at[0]], rows[nb], gsem[nb])
            pltpu.async_copy(D_hbm.at[idxs[nb].at[1]], advs[nb], dsem[nb])
            compute(b)
            pltpu.async_copy(rows[b], acc.at[idxs[b].at[1]], ssem[b], add=True)
        return carry

    lax.fori_loop(0, NCH // 2, pair, 0)
    pltpu.make_async_copy(T_hbm.at[idx_a.at[0]], rows_a, ga).wait()
    pltpu.make_async_copy(D_hbm.at[idx_a.at[1]], adv_a, da).wait()
    pltpu.make_async_copy(rows_b, acc.at[idx_b.at[1]], sb).wait()
    plsc.subcore_barrier()
    pltpu.sync_copy(acc.at[pl.ds(s * RPT, RPT)],
                    out_hbm.at[c, pl.ds(s * RPT, RPT)])


@functools.lru_cache(maxsize=None)
def _make_sc_pass(width):
    mesh = plsc.VectorSubcoreMesh(core_axis_name="c", subcore_axis_name="s")
    return functools.partial(
        pl.kernel,
        out_type=jax.ShapeDtypeStruct((2, NPAD, width), _f32),
        mesh=mesh,
        scratch_types=[
            pltpu.VMEM_SHARED((NPAD, width), _f32),
            pltpu.VMEM((2, K), _i32),
            pltpu.VMEM((2, K), _i32),
            pltpu.VMEM((K, width), _f32),
            pltpu.VMEM((K, width), _f32),
            pltpu.VMEM((K, 16), _f32),
            pltpu.VMEM((K, 16), _f32),
            pltpu.VMEM((ZR, width), _f32),
            pltpu.SemaphoreType.DMA,
            pltpu.SemaphoreType.DMA,
            pltpu.SemaphoreType.DMA,
            pltpu.SemaphoreType.DMA,
            pltpu.SemaphoreType.DMA,
            pltpu.SemaphoreType.DMA,
            pltpu.SemaphoreType.DMA,
            pltpu.SemaphoreType.DMA,
        ],
        compiler_params=pltpu.CompilerParams(use_tc_tiling_on_sc=False,
                                             needs_layout_passes=False),
    )(functools.partial(_sc_body, width))


def _sc_wide(t, d, e3):
    return _make_sc_pass(WR)(t, d, e3)


def _sc_narrow(t, d, e3):
    return _make_sc_pass(WN)(t, d, e3)


def _prep1_body(x_ref, w_ref, as_ref, ad_ref, t_ref, d_ref):
    h = x_ref[...] * w_ref[...]
    t = h * as_ref[...]
    s0 = jnp.sum(t[:, :64], axis=1, keepdims=True)
    s1 = jnp.sum(t[:, 64:], axis=1, keepdims=True)
    u = h * ad_ref[...]
    d0 = jnp.sum(u[:, :64], axis=1, keepdims=True)
    d1 = jnp.sum(u[:, 64:], axis=1, keepdims=True)
    zpad = jnp.zeros((h.shape[0], 14), _f32)
    t_ref[...] = jnp.concatenate([h, s0, s1, zpad], axis=1)
    d_ref[...] = jnp.concatenate([d0, d1, zpad], axis=1)


def _tc_prep1(x_pad, w1, asf, adf):
    grid = (NPAD // BLK,)
    return pl.pallas_call(
        _prep1_body,
        grid=grid,
        in_specs=[
            pl.BlockSpec((BLK, 1), lambda i: (i, 0)),
            pl.BlockSpec((1, 128), lambda i: (0, 0)),
            pl.BlockSpec((1, 128), lambda i: (0, 0)),
            pl.BlockSpec((1, 128), lambda i: (0, 0)),
        ],
        out_specs=[
            pl.BlockSpec((BLK, WR), lambda i: (i, 0)),
            pl.BlockSpec((BLK, WN), lambda i: (i, 0)),
        ],
        out_shape=[
            jax.ShapeDtypeStruct((NPAD, WR), _f32),
            jax.ShapeDtypeStruct((NPAD, WN), _f32),
        ],
    )(x_pad, w1, asf, adf)


def _combine_body(p0_ref, p1_ref, b_ref, w_ref, as_ref, ad_ref, t_ref, d_ref):
    p0 = p0_ref[...]
    p1 = p1_ref[...]
    num = p0[:, :128] + p1[:, :128]
    den0 = p0[:, 128:129] + p1[:, 128:129]
    den1 = p0[:, 129:130] + p1[:, 129:130]
    B = num.shape[0]
    den = jnp.concatenate([jnp.broadcast_to(den0, (B, 64)),
                           jnp.broadcast_to(den1, (B, 64))], axis=1)
    y = num / (den + 1e-16) + b_ref[...]
    y = jnp.maximum(y, 0.0)
    h = jnp.dot(y, w_ref[...], preferred_element_type=_f32)
    t = h * as_ref[...]
    s0 = jnp.sum(t[:, :64], axis=1, keepdims=True)
    s1 = jnp.sum(t[:, 64:], axis=1, keepdims=True)
    u = h * ad_ref[...]
    d0 = jnp.sum(u[:, :64], axis=1, keepdims=True)
    d1 = jnp.sum(u[:, 64:], axis=1, keepdims=True)
    zpad = jnp.zeros((B, 14), _f32)
    t_ref[...] = jnp.concatenate([h, s0, s1, zpad], axis=1)
    d_ref[...] = jnp.concatenate([d0, d1, zpad], axis=1)


def _tc_combine2(p0, p1, b1, w2, asf, adf):
    grid = (NPAD // BLK,)
    return pl.pallas_call(
        _combine_body,
        grid=grid,
        in_specs=[
            pl.BlockSpec((BLK, WR), lambda i: (i, 0)),
            pl.BlockSpec((BLK, WR), lambda i: (i, 0)),
            pl.BlockSpec((1, 128), lambda i: (0, 0)),
            pl.BlockSpec((128, 128), lambda i: (0, 0)),
            pl.BlockSpec((1, 128), lambda i: (0, 0)),
            pl.BlockSpec((1, 128), lambda i: (0, 0)),
        ],
        out_specs=[
            pl.BlockSpec((BLK, WR), lambda i: (i, 0)),
            pl.BlockSpec((BLK, WN), lambda i: (i, 0)),
        ],
        out_shape=[
            jax.ShapeDtypeStruct((NPAD, WR), _f32),
            jax.ShapeDtypeStruct((NPAD, WN), _f32),
        ],
    )(p0, p1, b1, w2, asf, adf)


def _combine3_body(p0_ref, p1_ref, b_ref, w_ref, as_ref, ad_ref, t_ref, d_ref):
    p0 = p0_ref[...]
    p1 = p1_ref[...]
    num = p0[:, :128] + p1[:, :128]
    den0 = p0[:, 128:129] + p1[:, 128:129]
    den1 = p0[:, 129:130] + p1[:, 129:130]
    B = num.shape[0]
    den = jnp.concatenate([jnp.broadcast_to(den0, (B, 64)),
                           jnp.broadcast_to(den1, (B, 64))], axis=1)
    y = num / (den + 1e-16) + b_ref[...]
    y = jnp.maximum(y, 0.0)
    h = jnp.dot(y, w_ref[...], preferred_element_type=_f32)
    as_s = as_ref[0, 0]
    ad_s = ad_ref[0, 0]
    t_ref[...] = jnp.concatenate([h, h * as_s, jnp.zeros((B, 14), _f32)],
                                 axis=1)
    d_ref[...] = jnp.concatenate([h * ad_s, jnp.zeros((B, 15), _f32)], axis=1)


def _tc_combine3(p0, p1, b2, w3, as3, ad3):
    grid = (NPAD // BLK,)
    return pl.pallas_call(
        _combine3_body,
        grid=grid,
        in_specs=[
            pl.BlockSpec((BLK, WR), lambda i: (i, 0)),
            pl.BlockSpec((BLK, WR), lambda i: (i, 0)),
            pl.BlockSpec((1, 128), lambda i: (0, 0)),
            pl.BlockSpec((128, 1), lambda i: (0, 0)),
            pl.BlockSpec((1, 1), lambda i: (0, 0)),
            pl.BlockSpec((1, 1), lambda i: (0, 0)),
        ],
        out_specs=[
            pl.BlockSpec((BLK, WN), lambda i: (i, 0)),
            pl.BlockSpec((BLK, WN), lambda i: (i, 0)),
        ],
        out_shape=[
            jax.ShapeDtypeStruct((NPAD, WN), _f32),
            jax.ShapeDtypeStruct((NPAD, WN), _f32),
        ],
    )(p0, p1, b2, w3, as3, ad3)


def _final_body(q0_ref, q1_ref, b_ref, o_ref):
    q0 = q0_ref[...]
    q1 = q1_ref[...]
    num = q0[:, 0:1] + q1[:, 0:1]
    den = q0[:, 1:2] + q1[:, 1:2]
    o_ref[...] = jax.nn.sigmoid(num / (den + 1e-16) + b_ref[...])


def _tc_final(q0, q1, b3):
    grid = (NPAD // BLK,)
    return pl.pallas_call(
        _final_body,
        grid=grid,
        in_specs=[
            pl.BlockSpec((BLK, WN), lambda i: (i, 0)),
            pl.BlockSpec((BLK, WN), lambda i: (i, 0)),
            pl.BlockSpec((1, 1), lambda i: (0, 0)),
        ],
        out_specs=pl.BlockSpec((BLK, 1), lambda i: (i, 0)),
        out_shape=jax.ShapeDtypeStruct((NPAD, 1), _f32),
    )(q0, q1, b3)


def kernel(x, edge_index, batch, W1, as1, ad1, b1, W2, as2, ad2, b2,
           W3, as3, ad3, b3):
    del batch
    loop = jnp.arange(N, dtype=edge_index.dtype)
    pad = jnp.full((EPAD - E,), N, dtype=edge_index.dtype)
    dummy = jnp.full((NW, 1, 2, K), N, dtype=edge_index.dtype)
    src3 = jnp.concatenate([edge_index[0], loop, pad]).reshape(NW, NCH, 1, K)
    dst3 = jnp.concatenate([edge_index[1], loop, pad]).reshape(NW, NCH, 1, K)
    e3 = jnp.concatenate(
        [jnp.concatenate([src3, dst3], axis=2), dummy], axis=1)
    x_pad = jnp.pad(x, ((0, NPAD - N), (0, 0)))

    t1, d1 = _tc_prep1(x_pad, W1.reshape(1, 128), as1.reshape(1, 128),
                       ad1.reshape(1, 128))
    p1 = _sc_wide(t1, d1, e3)
    t2, d2 = _tc_combine2(p1[0], p1[1], b1.reshape(1, 128), W2,
                          as2.reshape(1, 128), ad2.reshape(1, 128))
    p2 = _sc_wide(t2, d2, e3)
    t3, d3 = _tc_combine3(p2[0], p2[1], b2.reshape(1, 128), W3,
                          as3.reshape(1, 1), ad3.reshape(1, 1))
    p3 = _sc_narrow(t3, d3, e3)
    out = _tc_final(p3[0], p3[1], b3.reshape(1, 1))
    return out[:N]


# speedup vs baseline: 1.0445x; 1.0018x over previous
import functools

import jax
import jax.numpy as jnp
from jax import lax
from jax.experimental import pallas as pl
from jax.experimental.pallas import tpu as pltpu
from jax.experimental.pallas import tpu_sc as plsc

N = 10000
NPAD = 10240
E_RAW = 640000
E = E_RAW + N
NW = 32
K = 96
EPT = 20352
NCH = EPT // K
NCHP = NCH + 1
EPAD = NW * EPT
WR = 144
WN = 16
RPT = NPAD // 16
ZR = 40
BLK = 512

_f32 = jnp.float32
_i32 = jnp.int32


def _sc_body(width, T_hbm, D_hbm, e_hbm, out_hbm,
             acc, idx_a, idx_b, rows_a, rows_b, adv_a, adv_b,
             zbuf, ga, gb, da, db, sa, sb):
    c = lax.axis_index("c")
    s = lax.axis_index("s")
    wid = s * 2 + c
    iota16 = lax.iota(_i32, 16)
    z16 = jnp.zeros((16,), _f32)
    zi16 = jnp.zeros((16,), _i32)
    ng = width // 16
    idxs = [idx_a, idx_b]
    rows = [rows_a, rows_b]
    advs = [adv_a, adv_b]
    gsem = [ga, gb]
    dsem = [da, db]
    ssem = [sa, sb]

    for r in range(ZR):
        for q in range(ng):
            zbuf[r, pl.ds(q * 16, 16)] = z16
    for bb in range(RPT // ZR):
        pltpu.sync_copy(zbuf, acc.at[pl.ds(s * RPT + bb * ZR, ZR)])
    plsc.subcore_barrier()

    c128 = jnp.full((16,), 128, _i32)
    c129 = jnp.full((16,), 129, _i32)
    c1 = jnp.full((16,), 1, _i32)

    def compute(b):
        rows_v = rows[b]
        adv = advs[b]
        if width == WR:
            for g in range(K // 16):
                ids = iota16 + g * 16
                as0 = plsc.load_gather(rows_v, [ids, c128])
                as1 = plsc.load_gather(rows_v, [ids, c129])
                ad0 = plsc.load_gather(adv, [ids, zi16])
                ad1 = plsc.load_gather(adv, [ids, c1])
                s0 = as0 + ad0
                s1 = as1 + ad1
                w0 = jnp.exp(jnp.where(s0 > 0, s0, 0.2 * s0))
                w1 = jnp.exp(jnp.where(s1 > 0, s1, 0.2 * s1))
                for j in range(16):
                    e = g * 16 + j
                    w0b = jnp.full((16,), w0[j], _f32)
                    w1b = jnp.full((16,), w1[j], _f32)
                    for q in range(4):
                        rows_v[e, pl.ds(q * 16, 16)] = (
                            rows_v[e, pl.ds(q * 16, 16)] * w0b)
                    for q in range(4, 8):
                        rows_v[e, pl.ds(q * 16, 16)] = (
                            rows_v[e, pl.ds(q * 16, 16)] * w1b)
                    rows_v[e, pl.ds(128, 16)] = jnp.where(
                        iota16 == 0, w0b, jnp.where(iota16 == 1, w1b, z16))
        else:
            for g in range(K // 16):
                ids = iota16 + g * 16
                hv = plsc.load_gather(rows_v, [ids, zi16])
                asv = plsc.load_gather(rows_v, [ids, c1])
                adv0 = plsc.load_gather(adv, [ids, zi16])
                s0 = asv + adv0
                w = jnp.exp(jnp.where(s0 > 0, s0, 0.2 * s0))
                nm = w * hv
                for j in range(16):
                    e = g * 16 + j
                    nb = jnp.full((16,), nm[j], _f32)
                    wb = jnp.full((16,), w[j], _f32)
                    rows_v[e, pl.ds(0, 16)] = jnp.where(
                        iota16 == 0, nb, jnp.where(iota16 == 1, wb, z16))

    for q in range(K // 16):
        idx_b[1, pl.ds(q * 16, 16)] = zi16
    for r in range(K):
        for q in range(ng):
            rows_b[r, pl.ds(q * 16, 16)] = z16
    pltpu.async_copy(rows_b, acc.at[idx_b.at[1]], sb, add=True)
    pltpu.sync_copy(e_hbm.at[wid, 0], idx_a)
    pltpu.async_copy(T_hbm.at[idx_a.at[0]], rows_a, ga)
    pltpu.async_copy(D_hbm.at[idx_a.at[1]], adv_a, da)

    def pair(p, carry):
        for b in (0, 1):
            ci = 2 * p + b
            nb = 1 - b
            pltpu.make_async_copy(rows[nb], acc.at[idxs[nb].at[1]],
                                  ssem[nb]).wait()
            pltpu.sync_copy(e_hbm.at[wid, ci + 1], idxs[nb])
            pltpu.async_copy(T_hbm.at[idxs[nb].at[0]], rows[nb], gsem[nb])
            pltpu.async_copy(D_hbm.at[idxs[nb].at[1]], advs[nb], dsem[nb])
            pltpu.make_async_copy(T_hbm.at[idxs[b].at[0]], rows[b],
                                  gsem[b]).wait()
            pltpu.make_async_copy(D_hbm.at[idxs[b].at[1]], advs[b],
                                  dsem[b]).wait()
            compute(b)
            pltpu.async_copy(rows[b], acc.at[idxs[b].at[1]], ssem[b], add=True)
        return carry

    lax.fori_loop(0, NCH // 2, pair, 0)
    pltpu.make_async_copy(T_hbm.at[idx_a.at[0]], rows_a, ga).wait()
    pltpu.make_async_copy(D_hbm.at[idx_a.at[1]], adv_a, da).wait()
    pltpu.make_async_copy(rows_b, acc.at[idx_b.at[1]], sb).wait()
    plsc.subcore_barrier()
    pltpu.sync_copy(acc.at[pl.ds(s * RPT, RPT)],
                    out_hbm.at[c, pl.ds(s * RPT, RPT)])


@functools.lru_cache(maxsize=None)
def _make_sc_pass(width):
    mesh = plsc.VectorSubcoreMesh(core_axis_name="c", subcore_axis_name="s")
    return functools.partial(
        pl.kernel,
        out_type=jax.ShapeDtypeStruct((2, NPAD, width), _f32),
        mesh=mesh,
        scratch_types=[
            pltpu.VMEM_SHARED((NPAD, width), _f32),
            pltpu.VMEM((2, K), _i32),
            pltpu.VMEM((2, K), _i32),
            pltpu.VMEM((K, width), _f32),
            pltpu.VMEM((K, width), _f32),
            pltpu.VMEM((K, 16), _f32),
            pltpu.VMEM((K, 16), _f32),
            pltpu.VMEM((ZR, width), _f32),
            pltpu.SemaphoreType.DMA,
            pltpu.SemaphoreType.DMA,
            pltpu.SemaphoreType.DMA,
            pltpu.SemaphoreType.DMA,
            pltpu.SemaphoreType.DMA,
            pltpu.SemaphoreType.DMA,
        ],
        compiler_params=pltpu.CompilerParams(use_tc_tiling_on_sc=False,
                                             needs_layout_passes=False),
    )(functools.partial(_sc_body, width))


def _sc_wide(t, d, e3):
    return _make_sc_pass(WR)(t, d, e3)


def _sc_narrow(t, d, e3):
    return _make_sc_pass(WN)(t, d, e3)


def _prep1_body(x_ref, w_ref, as_ref, ad_ref, t_ref, d_ref):
    h = x_ref[...] * w_ref[...]
    t = h * as_ref[...]
    s0 = jnp.sum(t[:, :64], axis=1, keepdims=True)
    s1 = jnp.sum(t[:, 64:], axis=1, keepdims=True)
    u = h * ad_ref[...]
    d0 = jnp.sum(u[:, :64], axis=1, keepdims=True)
    d1 = jnp.sum(u[:, 64:], axis=1, keepdims=True)
    zpad = jnp.zeros((h.shape[0], 14), _f32)
    t_ref[...] = jnp.concatenate([h, s0, s1, zpad], axis=1)
    d_ref[...] = jnp.concatenate([d0, d1, zpad], axis=1)


def _tc_prep1(x_pad, w1, asf, adf):
    grid = (NPAD // BLK,)
    return pl.pallas_call(
        _prep1_body,
        grid=grid,
        in_specs=[
            pl.BlockSpec((BLK, 1), lambda i: (i, 0)),
            pl.BlockSpec((1, 128), lambda i: (0, 0)),
            pl.BlockSpec((1, 128), lambda i: (0, 0)),
            pl.BlockSpec((1, 128), lambda i: (0, 0)),
        ],
        out_specs=[
            pl.BlockSpec((BLK, WR), lambda i: (i, 0)),
            pl.BlockSpec((BLK, WN), lambda i: (i, 0)),
        ],
        out_shape=[
            jax.ShapeDtypeStruct((NPAD, WR), _f32),
            jax.ShapeDtypeStruct((NPAD, WN), _f32),
        ],
    )(x_pad, w1, asf, adf)


def _combine_body(p0_ref, p1_ref, b_ref, w_ref, as_ref, ad_ref, t_ref, d_ref):
    p0 = p0_ref[...]
    p1 = p1_ref[...]
    num = p0[:, :128] + p1[:, :128]
    den0 = p0[:, 128:129] + p1[:, 128:129]
    den1 = p0[:, 129:130] + p1[:, 129:130]
    B = num.shape[0]
    den = jnp.concatenate([jnp.broadcast_to(den0, (B, 64)),
                           jnp.broadcast_to(den1, (B, 64))], axis=1)
    y = num / (den + 1e-16) + b_ref[...]
    y = jnp.maximum(y, 0.0)
    h = jnp.dot(y, w_ref[...], preferred_element_type=_f32)
    t = h * as_ref[...]
    s0 = jnp.sum(t[:, :64], axis=1, keepdims=True)
    s1 = jnp.sum(t[:, 64:], axis=1, keepdims=True)
    u = h * ad_ref[...]
    d0 = jnp.sum(u[:, :64], axis=1, keepdims=True)
    d1 = jnp.sum(u[:, 64:], axis=1, keepdims=True)
    zpad = jnp.zeros((B, 14), _f32)
    t_ref[...] = jnp.concatenate([h, s0, s1, zpad], axis=1)
    d_ref[...] = jnp.concatenate([d0, d1, zpad], axis=1)


def _tc_combine2(p0, p1, b1, w2, asf, adf):
    grid = (NPAD // BLK,)
    return pl.pallas_call(
        _combine_body,
        grid=grid,
        in_specs=[
            pl.BlockSpec((BLK, WR), lambda i: (i, 0)),
            pl.BlockSpec((BLK, WR), lambda i: (i, 0)),
            pl.BlockSpec((1, 128), lambda i: (0, 0)),
            pl.BlockSpec((128, 128), lambda i: (0, 0)),
            pl.BlockSpec((1, 128), lambda i: (0, 0)),
            pl.BlockSpec((1, 128), lambda i: (0, 0)),
        ],
        out_specs=[
            pl.BlockSpec((BLK, WR), lambda i: (i, 0)),
            pl.BlockSpec((BLK, WN), lambda i: (i, 0)),
        ],
        out_shape=[
            jax.ShapeDtypeStruct((NPAD, WR), _f32),
            jax.ShapeDtypeStruct((NPAD, WN), _f32),
        ],
    )(p0, p1, b1, w2, asf, adf)


def _combine3_body(p0_ref, p1_ref, b_ref, w_ref, as_ref, ad_ref, t_ref, d_ref):
    p0 = p0_ref[...]
    p1 = p1_ref[...]
    num = p0[:, :128] + p1[:, :128]
    den0 = p0[:, 128:129] + p1[:, 128:129]
    den1 = p0[:, 129:130] + p1[:, 129:130]
    B = num.shape[0]
    den = jnp.concatenate([jnp.broadcast_to(den0, (B, 64)),
                           jnp.broadcast_to(den1, (B, 64))], axis=1)
    y = num / (den + 1e-16) + b_ref[...]
    y = jnp.maximum(y, 0.0)
    h = jnp.dot(y, w_ref[...], preferred_element_type=_f32)
    as_s = as_ref[0, 0]
    ad_s = ad_ref[0, 0]
    t_ref[...] = jnp.concatenate([h, h * as_s, jnp.zeros((B, 14), _f32)],
                                 axis=1)
    d_ref[...] = jnp.concatenate([h * ad_s, jnp.zeros((B, 15), _f32)], axis=1)


def _tc_combine3(p0, p1, b2, w3, as3, ad3):
    grid = (NPAD // BLK,)
    return pl.pallas_call(
        _combine3_body,
        grid=grid,
        in_specs=[
            pl.BlockSpec((BLK, WR), lambda i: (i, 0)),
            pl.BlockSpec((BLK, WR), lambda i: (i, 0)),
            pl.BlockSpec((1, 128), lambda i: (0, 0)),
            pl.BlockSpec((128, 1), lambda i: (0, 0)),
            pl.BlockSpec((1, 1), lambda i: (0, 0)),
            pl.BlockSpec((1, 1), lambda i: (0, 0)),
        ],
        out_specs=[
            pl.BlockSpec((BLK, WN), lambda i: (i, 0)),
            pl.BlockSpec((BLK, WN), lambda i: (i, 0)),
        ],
        out_shape=[
            jax.ShapeDtypeStruct((NPAD, WN), _f32),
            jax.ShapeDtypeStruct((NPAD, WN), _f32),
        ],
    )(p0, p1, b2, w3, as3, ad3)


def _final_body(q0_ref, q1_ref, b_ref, o_ref):
    q0 = q0_ref[...]
    q1 = q1_ref[...]
    num = q0[:, 0:1] + q1[:, 0:1]
    den = q0[:, 1:2] + q1[:, 1:2]
    o_ref[...] = jax.nn.sigmoid(num / (den + 1e-16) + b_ref[...])


def _tc_final(q0, q1, b3):
    grid = (NPAD // BLK,)
    return pl.pallas_call(
        _final_body,
        grid=grid,
        in_specs=[
            pl.BlockSpec((BLK, WN), lambda i: (i, 0)),
            pl.BlockSpec((BLK, WN), lambda i: (i, 0)),
            pl.BlockSpec((1, 1), lambda i: (0, 0)),
        ],
        out_specs=pl.BlockSpec((BLK, 1), lambda i: (i, 0)),
        out_shape=jax.ShapeDtypeStruct((NPAD, 1), _f32),
    )(q0, q1, b3)


def kernel(x, edge_index, batch, W1, as1, ad1, b1, W2, as2, ad2, b2,
           W3, as3, ad3, b3):
    del batch
    loop = jnp.arange(N, dtype=edge_index.dtype)
    pad = jnp.full((EPAD - E,), N, dtype=edge_index.dtype)
    dummy = jnp.full((NW, 1, 2, K), N, dtype=edge_index.dtype)
    src3 = jnp.concatenate([edge_index[0], loop, pad]).reshape(NW, NCH, 1, K)
    dst3 = jnp.concatenate([edge_index[1], loop, pad]).reshape(NW, NCH, 1, K)
    e3 = jnp.concatenate(
        [jnp.concatenate([src3, dst3], axis=2), dummy], axis=1)
    x_pad = jnp.pad(x, ((0, NPAD - N), (0, 0)))

    t1, d1 = _tc_prep1(x_pad, W1.reshape(1, 128), as1.reshape(1, 128),
                       ad1.reshape(1, 128))
    p1 = _sc_wide(t1, d1, e3)
    t2, d2 = _tc_combine2(p1[0], p1[1], b1.reshape(1, 128), W2,
                          as2.reshape(1, 128), ad2.reshape(1, 128))
    p2 = _sc_wide(t2, d2, e3)
    t3, d3 = _tc_combine3(p2[0], p2[1], b2.reshape(1, 128), W3,
                          as3.reshape(1, 1), ad3.reshape(1, 1))
    p3 = _sc_narrow(t3, d3, e3)
    out = _tc_final(p3[0], p3[1], b3.reshape(1, 1))
    return out[:N]
